# baseline (speedup 1.0000x reference)
import functools

import jax
import jax.numpy as jnp
from jax import lax
from jax.experimental import pallas as pl
from jax.experimental.pallas import tpu as pltpu
from jax.experimental.pallas import tpu_sc as plsc

K = 20
EPS = 1e-5
NEG = -3.4e38
B, N = 16, 1024


def _knn_kernel(xt_ref, xtf_ref, idx_ref, *, b0):
    b = pl.program_id(0) + b0
    xr = xt_ref[0]
    xf = xtf_ref[0]
    inner = -2.0 * jax.lax.dot_general(
        xr, xf, (((1,), (1,)), ((), ())), preferred_element_type=jnp.float32)
    xxr = jnp.sum(xr * xr, axis=1, keepdims=True)
    xxf = jnp.sum(xf * xf, axis=1)
    pd = -xxr - inner - xxf[None, :]
    R = pd.shape[0]
    iota = jax.lax.broadcasted_iota(jnp.int32, (R, N), 1)
    off = b * N
    sels = []
    for _ in range(K):
        sel = jnp.argmax(pd, axis=1).astype(jnp.int32)[:, None]
        sels.append(sel + off)
        pd = jnp.where(iota == sel, NEG, pd)
    idx_ref[0] = jnp.concatenate(sels, axis=1)


def _knn_idx_flat(xt, b0, nb, R=256):
    D = xt.shape[-1]
    return pl.pallas_call(
        functools.partial(_knn_kernel, b0=b0),
        grid=(nb, N // R),
        in_specs=[
            pl.BlockSpec((1, R, D), lambda b, i: (b + b0, i, 0)),
            pl.BlockSpec((1, N, D), lambda b, i: (b + b0, 0, 0)),
        ],
        out_specs=pl.BlockSpec((1, R, K), lambda b, i: (b, i, 0)),
        out_shape=jax.ShapeDtypeStruct((nb, N, K), jnp.int32),
    )(xt, xt)


def _edge_kernel(xt_ref, fg_ref, w_ref, m_ref, hsum_ref, hsq_ref, *, C):
    first = (pl.program_id(0) == 0) & (pl.program_id(1) == 0)
    xr = xt_ref[0][:, :C]
    fg = fg_ref[0][:, :, :C]
    Rn = xr.shape[0]
    xb = jnp.broadcast_to(xr[:, None, :], fg.shape)
    f2 = jnp.concatenate([xb, fg - xb], axis=-1)
    f2 = f2.reshape(Rn * K, 2 * C)
    h = jax.lax.dot_general(
        f2, w_ref[...], (((1,), (1,)), ((), ())),
        preferred_element_type=jnp.float32)
    s = jnp.sum(h, axis=0)
    q = jnp.sum(h * h, axis=0)
    m = jnp.max(h.reshape(Rn, K, -1), axis=1)

    @pl.when(first)
    def _():
        hsum_ref[...] = jnp.zeros_like(hsum_ref)
        hsq_ref[...] = jnp.zeros_like(hsq_ref)

    hsum_ref[...] += s[None, :]
    hsq_ref[...] += q[None, :]
    m_ref[0] = m


def _edge_mm(xt, fg, W, b0, nb, R=128):
    O, C2 = W.shape
    C = C2 // 2
    Dp = xt.shape[-1]
    return pl.pallas_call(
        functools.partial(_edge_kernel, C=C),
        grid=(nb, N // R),
        in_specs=[
            pl.BlockSpec((1, R, Dp), lambda b, i: (b + b0, i, 0)),
            pl.BlockSpec((1, R, K, Dp), lambda b, i: (b, i, 0, 0)),
            pl.BlockSpec((O, C2), lambda b, i: (0, 0)),
        ],
        out_specs=[
            pl.BlockSpec((1, R, O), lambda b, i: (b, i, 0)),
            pl.BlockSpec((1, O), lambda b, i: (0, 0)),
            pl.BlockSpec((1, O), lambda b, i: (0, 0)),
        ],
        out_shape=[
            jax.ShapeDtypeStruct((nb, N, O), jnp.float32),
            jax.ShapeDtypeStruct((1, O), jnp.float32),
            jax.ShapeDtypeStruct((1, O), jnp.float32),
        ],
    )(xt, fg, W)


def _norm_kernel(m_ref, hsum_ref, hsq_ref, g_ref, b_ref, o_ref, *, M, Dp):
    mean = hsum_ref[0] / M
    var = hsq_ref[0] / M - mean * mean
    h = ((m_ref[0] - mean[None, :]) / jnp.sqrt(var + EPS)[None, :]
         * g_ref[...][None, :] + b_ref[...][None, :])
    h = jnp.where(h >= 0, h, 0.2 * h)
    O = h.shape[-1]
    if Dp > O:
        h = jnp.pad(h, ((0, 0), (0, Dp - O)))
    o_ref[0] = h


def _bn_lrelu(m, hsum, hsq, g, b, Dp=None):
    O = m.shape[-1]
    Dp = Dp or O
    return pl.pallas_call(
        functools.partial(_norm_kernel, M=float(B * N * K), Dp=Dp),
        grid=(B,),
        in_specs=[
            pl.BlockSpec((1, N, O), lambda i: (i, 0, 0)),
            pl.BlockSpec((1, O), lambda i: (0, 0)),
            pl.BlockSpec((1, O), lambda i: (0, 0)),
            pl.BlockSpec((O,), lambda i: (0,)),
            pl.BlockSpec((O,), lambda i: (0,)),
        ],
        out_specs=pl.BlockSpec((1, N, Dp), lambda i: (i, 0, 0)),
        out_shape=jax.ShapeDtypeStruct((B, N, Dp), jnp.float32),
    )(m, hsum, hsq, g, b)


_NW = 32
_CH = 128


def _gather(xt, idx_flat):
    Dp = xt.shape[-1]
    nb = idx_flat.shape[0]
    rows = nb * N * K
    per = rows // _NW
    table = xt.reshape(B * N, Dp)
    idx1 = idx_flat.reshape(rows)
    mesh = plsc.VectorSubcoreMesh(core_axis_name="c", subcore_axis_name="s")

    @functools.partial(
        pl.kernel, mesh=mesh,
        compiler_params=pltpu.CompilerParams(use_tc_tiling_on_sc=False),
        out_type=jax.ShapeDtypeStruct((rows, Dp), jnp.float32),
        scratch_types=[
            pltpu.VMEM((_CH,), jnp.int32),
            pltpu.VMEM((_CH, Dp), jnp.float32),
            pltpu.VMEM((_CH,), jnp.int32),
            pltpu.VMEM((_CH, Dp), jnp.float32),
            pltpu.SemaphoreType.DMA,
            pltpu.SemaphoreType.DMA,
        ],
    )
    def gk(table_hbm, idx_hbm, out_hbm, idx_a, buf_a, idx_b, buf_b, sem_a,
           sem_b):
        wid = lax.axis_index("s") * 2 + lax.axis_index("c")
        base = wid * per
        nch = per // (2 * _CH)

        def body(i, _):
            off = base + i * 2 * _CH
            pltpu.sync_copy(idx_hbm.at[pl.ds(off, _CH)], idx_a)
            cp_a = pltpu.async_copy(table_hbm.at[idx_a], buf_a, sem_a)
            pltpu.sync_copy(idx_hbm.at[pl.ds(off + _CH, _CH)], idx_b)
            cp_b = pltpu.async_copy(table_hbm.at[idx_b], buf_b, sem_b)
            cp_a.wait()
            pltpu.sync_copy(buf_a, out_hbm.at[pl.ds(off, _CH)])
            cp_b.wait()
            pltpu.sync_copy(buf_b, out_hbm.at[pl.ds(off + _CH, _CH)])
            return 0

        lax.fori_loop(0, nch, body, 0)

    return gk(table, idx1).reshape(nb, N, K, Dp)


def _final_kernel(x1_ref, x2_ref, x3_ref, x4_ref, w_ref,
                  hsum_ref, hsq_ref, hmax_ref):
    i = pl.program_id(0)
    w = w_ref[...]
    h = jax.lax.dot_general(
        x1_ref[0], w[:, 0:64], (((1,), (1,)), ((), ())),
        preferred_element_type=jnp.float32)
    h += jax.lax.dot_general(
        x2_ref[0], w[:, 64:128], (((1,), (1,)), ((), ())),
        preferred_element_type=jnp.float32)
    h += jax.lax.dot_general(
        x3_ref[0], w[:, 128:256], (((1,), (1,)), ((), ())),
        preferred_element_type=jnp.float32)
    h += jax.lax.dot_general(
        x4_ref[0], w[:, 256:512], (((1,), (1,)), ((), ())),
        preferred_element_type=jnp.float32)
    s = jnp.sum(h, axis=0)
    q = jnp.sum(h * h, axis=0)
    m = jnp.max(h, axis=0)

    @pl.when(i == 0)
    def _():
        hsum_ref[...] = jnp.zeros_like(hsum_ref)
        hsq_ref[...] = jnp.zeros_like(hsq_ref)

    hsum_ref[...] += s[None, :]
    hsq_ref[...] += q[None, :]
    hmax_ref[pl.ds(i, 1), :] = m[None, :]


def _final_norm_kernel(hsum_ref, hsq_ref, hmax_ref, g_ref, b_ref, o_ref):
    M = float(B * N)
    mean = hsum_ref[0] / M
    var = hsq_ref[0] / M - mean * mean
    h = ((hmax_ref[...] - mean[None, :]) / jnp.sqrt(var + EPS)[None, :]
         * g_ref[...][None, :] + b_ref[...][None, :])
    o_ref[...] = jnp.where(h >= 0, h, 0.2 * h)


def _final_stage(x1, x2, x3, x4, W5, g5, b5):
    O = W5.shape[0]
    hsum, hsq, hmax = pl.pallas_call(
        _final_kernel,
        grid=(B,),
        in_specs=[
            pl.BlockSpec((1, N, 64), lambda i: (i, 0, 0)),
            pl.BlockSpec((1, N, 64), lambda i: (i, 0, 0)),
            pl.BlockSpec((1, N, 128), lambda i: (i, 0, 0)),
            pl.BlockSpec((1, N, 256), lambda i: (i, 0, 0)),
            pl.BlockSpec((O, 512), lambda i: (0, 0)),
        ],
        out_specs=[
            pl.BlockSpec((1, O), lambda i: (0, 0)),
            pl.BlockSpec((1, O), lambda i: (0, 0)),
            pl.BlockSpec((B, O), lambda i: (0, 0)),
        ],
        out_shape=[
            jax.ShapeDtypeStruct((1, O), jnp.float32),
            jax.ShapeDtypeStruct((1, O), jnp.float32),
            jax.ShapeDtypeStruct((B, O), jnp.float32),
        ],
    )(x1, x2, x3, x4, W5)
    return pl.pallas_call(
        _final_norm_kernel,
        out_shape=jax.ShapeDtypeStruct((B, O), jnp.float32),
    )(hsum, hsq, hmax, g5, b5)


def _layer(xt, W, g, b, Dp_out=None, waves=4):
    nb = B // waves
    ms, hsums, hsqs = [], [], []
    for w in range(waves):
        idx = _knn_idx_flat(xt, w * nb, nb)
        fg = _gather(xt, idx)
        m, hsum, hsq = _edge_mm(xt, fg, W, w * nb, nb)
        ms.append(m)
        hsums.append(hsum)
        hsqs.append(hsq)
    m = jnp.concatenate(ms, axis=0)
    hsum = functools.reduce(jnp.add, hsums)
    hsq = functools.reduce(jnp.add, hsqs)
    return _bn_lrelu(m, hsum, hsq, g, b, Dp=Dp_out)


def kernel(x, W1, W2, W3, W4, W5, g1, b1, g2, b2, g3, b3, g4, b4, g5, b5):
    xt0 = jnp.pad(jnp.transpose(x, (0, 2, 1)), ((0, 0), (0, 0), (0, 13)))
    x1 = _layer(xt0, W1, g1, b1)
    x2 = _layer(x1, W2, g2, b2)
    x3 = _layer(x2, W3, g3, b3)
    x4 = _layer(x3, W4, g4, b4)
    return _final_stage(x1, x2, x3, x4, W5, g5, b5)


# speedup vs baseline: 1.0031x; 1.0031x over previous
import functools

import jax
import jax.numpy as jnp
from jax import lax
from jax.experimental import pallas as pl
from jax.experimental.pallas import tpu as pltpu
from jax.experimental.pallas import tpu_sc as plsc

K = 20
EPS = 1e-5
NEG = -3.4e38
B, N = 16, 1024


def _norm_apply(m, hs, hq, g, bb):
    M = float(B * N * K)
    mean = hs / M
    var = hq / M - mean * mean
    h = (m - mean[None, :]) / jnp.sqrt(var + EPS)[None, :] \
        * g[None, :] + bb[None, :]
    return jnp.where(h >= 0, h, 0.2 * h)


def _knn_kernel(*refs, norm):
    if norm:
        (mt_ref, mf_ref, hs_ref, hq_ref, g_ref, b_ref,
         idx_ref, xt_ref) = refs
        hs, hq = hs_ref[0], hq_ref[0]
        g, bb = g_ref[...], b_ref[...]
        xr = _norm_apply(mt_ref[0], hs, hq, g, bb)
        xf = _norm_apply(mf_ref[0], hs, hq, g, bb)
        xt_ref[0] = xr
    else:
        mt_ref, mf_ref, idx_ref = refs
        xr = mt_ref[0]
        xf = mf_ref[0]
    inner = -2.0 * jax.lax.dot_general(
        xr, xf, (((1,), (1,)), ((), ())), preferred_element_type=jnp.float32)
    xxr = jnp.sum(xr * xr, axis=1, keepdims=True)
    xxf = jnp.sum(xf * xf, axis=1)
    pd = -xxr - inner - xxf[None, :]
    R = pd.shape[0]
    iota = jax.lax.broadcasted_iota(jnp.int32, (R, N), 1)
    off = pl.program_id(0) * N
    sels = []
    for _ in range(K):
        sel = jnp.argmax(pd, axis=1).astype(jnp.int32)[:, None]
        sels.append(sel + off)
        pd = jnp.where(iota == sel, NEG, pd)
    idx_ref[0] = jnp.concatenate(sels, axis=1)


def _knn_raw(xt, b0, nb, R=256):
    D = xt.shape[-1]
    return pl.pallas_call(
        functools.partial(_knn_kernel, norm=False),
        grid=(nb, N // R),
        in_specs=[
            pl.BlockSpec((1, R, D), lambda b, i: (b + b0, i, 0)),
            pl.BlockSpec((1, N, D), lambda b, i: (b + b0, 0, 0)),
        ],
        out_specs=pl.BlockSpec((1, R, K), lambda b, i: (b, i, 0)),
        out_shape=jax.ShapeDtypeStruct((nb, N, K), jnp.int32),
    )(xt, xt)


def _knn_norm(m, hsum, hsq, g, bb, b0, nb, R=256):
    D = m.shape[-1]
    return pl.pallas_call(
        functools.partial(_knn_kernel, norm=True),
        grid=(nb, N // R),
        in_specs=[
            pl.BlockSpec((1, R, D), lambda b, i: (b + b0, i, 0)),
            pl.BlockSpec((1, N, D), lambda b, i: (b + b0, 0, 0)),
            pl.BlockSpec((1, D), lambda b, i: (0, 0)),
            pl.BlockSpec((1, D), lambda b, i: (0, 0)),
            pl.BlockSpec((D,), lambda b, i: (0,)),
            pl.BlockSpec((D,), lambda b, i: (0,)),
        ],
        out_specs=[
            pl.BlockSpec((1, R, K), lambda b, i: (b, i, 0)),
            pl.BlockSpec((1, R, D), lambda b, i: (b, i, 0)),
        ],
        out_shape=[
            jax.ShapeDtypeStruct((nb, N, K), jnp.int32),
            jax.ShapeDtypeStruct((nb, N, D), jnp.float32),
        ],
    )(m, m, hsum, hsq, g, bb)


def _edge_kernel(xt_ref, fg_ref, w_ref, m_ref, hsum_ref, hsq_ref, *, C):
    first = (pl.program_id(0) == 0) & (pl.program_id(1) == 0)
    xr = xt_ref[0][:, :C]
    fg = fg_ref[0][:, :, :C]
    Rn = xr.shape[0]
    xb = jnp.broadcast_to(xr[:, None, :], fg.shape)
    f2 = jnp.concatenate([xb, fg - xb], axis=-1)
    f2 = f2.reshape(Rn * K, 2 * C)
    h = jax.lax.dot_general(
        f2, w_ref[...], (((1,), (1,)), ((), ())),
        preferred_element_type=jnp.float32)
    s = jnp.sum(h, axis=0)
    q = jnp.sum(h * h, axis=0)
    m = jnp.max(h.reshape(Rn, K, -1), axis=1)

    @pl.when(first)
    def _():
        hsum_ref[...] = jnp.zeros_like(hsum_ref)
        hsq_ref[...] = jnp.zeros_like(hsq_ref)

    hsum_ref[...] += s[None, :]
    hsq_ref[...] += q[None, :]
    m_ref[0] = m


def _edge_mm(xt, fg, W, b0, nb, R=128):
    O, C2 = W.shape
    C = C2 // 2
    Dp = xt.shape[-1]
    return pl.pallas_call(
        functools.partial(_edge_kernel, C=C),
        grid=(nb, N // R),
        in_specs=[
            pl.BlockSpec((1, R, Dp), lambda b, i: (b + b0, i, 0)),
            pl.BlockSpec((1, R, K, Dp), lambda b, i: (b, i, 0, 0)),
            pl.BlockSpec((O, C2), lambda b, i: (0, 0)),
        ],
        out_specs=[
            pl.BlockSpec((1, R, O), lambda b, i: (b, i, 0)),
            pl.BlockSpec((1, O), lambda b, i: (0, 0)),
            pl.BlockSpec((1, O), lambda b, i: (0, 0)),
        ],
        out_shape=[
            jax.ShapeDtypeStruct((nb, N, O), jnp.float32),
            jax.ShapeDtypeStruct((1, O), jnp.float32),
            jax.ShapeDtypeStruct((1, O), jnp.float32),
        ],
    )(xt, fg, W)


def _norm_kernel(m_ref, hsum_ref, hsq_ref, g_ref, b_ref, o_ref, *, M, Dp):
    mean = hsum_ref[0] / M
    var = hsq_ref[0] / M - mean * mean
    h = ((m_ref[0] - mean[None, :]) / jnp.sqrt(var + EPS)[None, :]
         * g_ref[...][None, :] + b_ref[...][None, :])
    h = jnp.where(h >= 0, h, 0.2 * h)
    O = h.shape[-1]
    if Dp > O:
        h = jnp.pad(h, ((0, 0), (0, Dp - O)))
    o_ref[0] = h


def _bn_lrelu(m, hsum, hsq, g, b, Dp=None):
    O = m.shape[-1]
    Dp = Dp or O
    return pl.pallas_call(
        functools.partial(_norm_kernel, M=float(B * N * K), Dp=Dp),
        grid=(B,),
        in_specs=[
            pl.BlockSpec((1, N, O), lambda i: (i, 0, 0)),
            pl.BlockSpec((1, O), lambda i: (0, 0)),
            pl.BlockSpec((1, O), lambda i: (0, 0)),
            pl.BlockSpec((O,), lambda i: (0,)),
            pl.BlockSpec((O,), lambda i: (0,)),
        ],
        out_specs=pl.BlockSpec((1, N, Dp), lambda i: (i, 0, 0)),
        out_shape=jax.ShapeDtypeStruct((B, N, Dp), jnp.float32),
    )(m, hsum, hsq, g, b)


_NW = 32
_CH = 128


def _gather(xt, idx_flat):
    Dp = xt.shape[-1]
    nb = idx_flat.shape[0]
    rows = nb * N * K
    per = rows // _NW
    table = xt.reshape(nb * N, Dp)
    idx1 = idx_flat.reshape(rows)
    mesh = plsc.VectorSubcoreMesh(core_axis_name="c", subcore_axis_name="s")

    @functools.partial(
        pl.kernel, mesh=mesh,
        compiler_params=pltpu.CompilerParams(use_tc_tiling_on_sc=False),
        out_type=jax.ShapeDtypeStruct((rows, Dp), jnp.float32),
        scratch_types=[
            pltpu.VMEM((_CH,), jnp.int32),
            pltpu.VMEM((_CH, Dp), jnp.float32),
            pltpu.VMEM((_CH,), jnp.int32),
            pltpu.VMEM((_CH, Dp), jnp.float32),
            pltpu.SemaphoreType.DMA,
            pltpu.SemaphoreType.DMA,
        ],
    )
    def gk(table_hbm, idx_hbm, out_hbm, idx_a, buf_a, idx_b, buf_b, sem_a,
           sem_b):
        wid = lax.axis_index("s") * 2 + lax.axis_index("c")
        base = wid * per
        nch = per // (2 * _CH)

        def body(i, _):
            off = base + i * 2 * _CH
            pltpu.sync_copy(idx_hbm.at[pl.ds(off, _CH)], idx_a)
            cp_a = pltpu.async_copy(table_hbm.at[idx_a], buf_a, sem_a)
            pltpu.sync_copy(idx_hbm.at[pl.ds(off + _CH, _CH)], idx_b)
            cp_b = pltpu.async_copy(table_hbm.at[idx_b], buf_b, sem_b)
            cp_a.wait()
            pltpu.sync_copy(buf_a, out_hbm.at[pl.ds(off, _CH)])
            cp_b.wait()
            pltpu.sync_copy(buf_b, out_hbm.at[pl.ds(off + _CH, _CH)])
            return 0

        lax.fori_loop(0, nch, body, 0)

    return gk(table, idx1).reshape(nb, N, K, Dp)


def _final_kernel(x1_ref, x2_ref, x3_ref, x4_ref, w_ref,
                  hsum_ref, hsq_ref, hmax_ref):
    i = pl.program_id(0)
    w = w_ref[...]
    h = jax.lax.dot_general(
        x1_ref[0], w[:, 0:64], (((1,), (1,)), ((), ())),
        preferred_element_type=jnp.float32)
    h += jax.lax.dot_general(
        x2_ref[0], w[:, 64:128], (((1,), (1,)), ((), ())),
        preferred_element_type=jnp.float32)
    h += jax.lax.dot_general(
        x3_ref[0], w[:, 128:256], (((1,), (1,)), ((), ())),
        preferred_element_type=jnp.float32)
    h += jax.lax.dot_general(
        x4_ref[0], w[:, 256:512], (((1,), (1,)), ((), ())),
        preferred_element_type=jnp.float32)
    s = jnp.sum(h, axis=0)
    q = jnp.sum(h * h, axis=0)
    m = jnp.max(h, axis=0)

    @pl.when(i == 0)
    def _():
        hsum_ref[...] = jnp.zeros_like(hsum_ref)
        hsq_ref[...] = jnp.zeros_like(hsq_ref)

    hsum_ref[...] += s[None, :]
    hsq_ref[...] += q[None, :]
    hmax_ref[pl.ds(i, 1), :] = m[None, :]


def _final_norm_kernel(hsum_ref, hsq_ref, hmax_ref, g_ref, b_ref, o_ref):
    M = float(B * N)
    mean = hsum_ref[0] / M
    var = hsq_ref[0] / M - mean * mean
    h = ((hmax_ref[...] - mean[None, :]) / jnp.sqrt(var + EPS)[None, :]
         * g_ref[...][None, :] + b_ref[...][None, :])
    o_ref[...] = jnp.where(h >= 0, h, 0.2 * h)


def _final_stage(x1, x2, x3, x4, W5, g5, b5):
    O = W5.shape[0]
    hsum, hsq, hmax = pl.pallas_call(
        _final_kernel,
        grid=(B,),
        in_specs=[
            pl.BlockSpec((1, N, 64), lambda i: (i, 0, 0)),
            pl.BlockSpec((1, N, 64), lambda i: (i, 0, 0)),
            pl.BlockSpec((1, N, 128), lambda i: (i, 0, 0)),
            pl.BlockSpec((1, N, 256), lambda i: (i, 0, 0)),
            pl.BlockSpec((O, 512), lambda i: (0, 0)),
        ],
        out_specs=[
            pl.BlockSpec((1, O), lambda i: (0, 0)),
            pl.BlockSpec((1, O), lambda i: (0, 0)),
            pl.BlockSpec((B, O), lambda i: (0, 0)),
        ],
        out_shape=[
            jax.ShapeDtypeStruct((1, O), jnp.float32),
            jax.ShapeDtypeStruct((1, O), jnp.float32),
            jax.ShapeDtypeStruct((B, O), jnp.float32),
        ],
    )(x1, x2, x3, x4, W5)
    return pl.pallas_call(
        _final_norm_kernel,
        out_shape=jax.ShapeDtypeStruct((B, O), jnp.float32),
    )(hsum, hsq, hmax, g5, b5)


def _layer(prev, W, waves=2):
    nb = B // waves
    ms, hsums, hsqs, xts = [], [], [], []
    for w in range(waves):
        if prev[0] == "raw":
            xt_w = jax.lax.slice_in_dim(prev[1], w * nb, (w + 1) * nb, axis=0)
            idx = _knn_raw(prev[1], w * nb, nb)
        else:
            _, m_in, hs_in, hq_in, g_in, b_in = prev
            idx, xt_w = _knn_norm(m_in, hs_in, hq_in, g_in, b_in, w * nb, nb)
        fg = _gather(xt_w, idx)
        m, hsum, hsq = _edge_mm(xt_w, fg, W, 0, nb)
        ms.append(m)
        hsums.append(hsum)
        hsqs.append(hsq)
        xts.append(xt_w)
    m = jnp.concatenate(ms, axis=0)
    hsum = functools.reduce(jnp.add, hsums)
    hsq = functools.reduce(jnp.add, hsqs)
    xt_full = jnp.concatenate(xts, axis=0) if prev[0] == "norm" else None
    return m, hsum, hsq, xt_full


def kernel(x, W1, W2, W3, W4, W5, g1, b1, g2, b2, g3, b3, g4, b4, g5, b5):
    xt0 = jnp.pad(jnp.transpose(x, (0, 2, 1)), ((0, 0), (0, 0), (0, 13)))
    m1, s1, q1, _ = _layer(("raw", xt0), W1)
    m2, s2, q2, x1 = _layer(("norm", m1, s1, q1, g1, b1), W2)
    m3, s3, q3, x2 = _layer(("norm", m2, s2, q2, g2, b2), W3)
    m4, s4, q4, x3 = _layer(("norm", m3, s3, q3, g3, b3), W4)
    x4 = _bn_lrelu(m4, s4, q4, g4, b4)
    return _final_stage(x1, x2, x3, x4, W5, g5, b5)


# speedup vs baseline: 1.0700x; 1.0667x over previous
import functools

import jax
import jax.numpy as jnp
from jax import lax
from jax.experimental import pallas as pl
from jax.experimental.pallas import tpu as pltpu
from jax.experimental.pallas import tpu_sc as plsc

K = 20
EPS = 1e-5
NEG = -3.4e38
B, N = 16, 1024


def _norm_apply(m, hs, hq, g, bb):
    M = float(B * N * K)
    mean = hs / M
    var = hq / M - mean * mean
    h = (m - mean[None, :]) / jnp.sqrt(var + EPS)[None, :] \
        * g[None, :] + bb[None, :]
    return jnp.where(h >= 0, h, 0.2 * h)


def _knn_kernel(*refs, norm):
    if norm:
        (mt_ref, mf_ref, hs_ref, hq_ref, g_ref, b_ref,
         idx_ref, xt_ref) = refs
        hs, hq = hs_ref[0], hq_ref[0]
        g, bb = g_ref[...], b_ref[...]
        xr = _norm_apply(mt_ref[0], hs, hq, g, bb)
        xf = _norm_apply(mf_ref[0], hs, hq, g, bb)
        xt_ref[0] = xr
    else:
        mt_ref, mf_ref, idx_ref = refs
        xr = mt_ref[0]
        xf = mf_ref[0]
    inner = -2.0 * jax.lax.dot_general(
        xr, xf, (((1,), (1,)), ((), ())), preferred_element_type=jnp.float32)
    xxr = jnp.sum(xr * xr, axis=1, keepdims=True)
    xxf = jnp.sum(xf * xf, axis=1)
    pd = -xxr - inner - xxf[None, :]
    R = pd.shape[0]
    iota = jax.lax.broadcasted_iota(jnp.int32, (R, N), 1)
    off = pl.program_id(0) * N
    sels = []
    for _ in range(K):
        sel = jnp.argmax(pd, axis=1).astype(jnp.int32)[:, None]
        sels.append(sel + off)
        pd = jnp.where(iota == sel, NEG, pd)
    idx_ref[0] = jnp.concatenate(sels, axis=1)


def _knn_raw(xt, b0, nb, R=512):
    D = xt.shape[-1]
    return pl.pallas_call(
        functools.partial(_knn_kernel, norm=False),
        grid=(nb, N // R),
        in_specs=[
            pl.BlockSpec((1, R, D), lambda b, i: (b + b0, i, 0)),
            pl.BlockSpec((1, N, D), lambda b, i: (b + b0, 0, 0)),
        ],
        out_specs=pl.BlockSpec((1, R, K), lambda b, i: (b, i, 0)),
        out_shape=jax.ShapeDtypeStruct((nb, N, K), jnp.int32),
    )(xt, xt)


def _knn_norm(m, hsum, hsq, g, bb, b0, nb, R=512):
    D = m.shape[-1]
    return pl.pallas_call(
        functools.partial(_knn_kernel, norm=True),
        grid=(nb, N // R),
        in_specs=[
            pl.BlockSpec((1, R, D), lambda b, i: (b + b0, i, 0)),
            pl.BlockSpec((1, N, D), lambda b, i: (b + b0, 0, 0)),
            pl.BlockSpec((1, D), lambda b, i: (0, 0)),
            pl.BlockSpec((1, D), lambda b, i: (0, 0)),
            pl.BlockSpec((D,), lambda b, i: (0,)),
            pl.BlockSpec((D,), lambda b, i: (0,)),
        ],
        out_specs=[
            pl.BlockSpec((1, R, K), lambda b, i: (b, i, 0)),
            pl.BlockSpec((1, R, D), lambda b, i: (b, i, 0)),
        ],
        out_shape=[
            jax.ShapeDtypeStruct((nb, N, K), jnp.int32),
            jax.ShapeDtypeStruct((nb, N, D), jnp.float32),
        ],
    )(m, m, hsum, hsq, g, bb)


def _edge_kernel(xt_ref, fg_ref, w_ref, m_ref, hsum_ref, hsq_ref, *, C):
    first = (pl.program_id(0) == 0) & (pl.program_id(1) == 0)
    xr = xt_ref[0][:, :C]
    fg = fg_ref[0][:, :, :C]
    Rn = xr.shape[0]
    xb = jnp.broadcast_to(xr[:, None, :], fg.shape)
    f2 = jnp.concatenate([xb, fg - xb], axis=-1)
    f2 = f2.reshape(Rn * K, 2 * C)
    h = jax.lax.dot_general(
        f2, w_ref[...], (((1,), (1,)), ((), ())),
        preferred_element_type=jnp.float32)
    s = jnp.sum(h, axis=0)
    q = jnp.sum(h * h, axis=0)
    m = jnp.max(h.reshape(Rn, K, -1), axis=1)

    @pl.when(first)
    def _():
        hsum_ref[...] = jnp.zeros_like(hsum_ref)
        hsq_ref[...] = jnp.zeros_like(hsq_ref)

    hsum_ref[...] += s[None, :]
    hsq_ref[...] += q[None, :]
    m_ref[0] = m


def _edge_mm(xt, fg, W, b0, nb, R=256):
    O, C2 = W.shape
    C = C2 // 2
    Dp = xt.shape[-1]
    return pl.pallas_call(
        functools.partial(_edge_kernel, C=C),
        grid=(nb, N // R),
        in_specs=[
            pl.BlockSpec((1, R, Dp), lambda b, i: (b + b0, i, 0)),
            pl.BlockSpec((1, R, K, Dp), lambda b, i: (b, i, 0, 0)),
            pl.BlockSpec((O, C2), lambda b, i: (0, 0)),
        ],
        out_specs=[
            pl.BlockSpec((1, R, O), lambda b, i: (b, i, 0)),
            pl.BlockSpec((1, O), lambda b, i: (0, 0)),
            pl.BlockSpec((1, O), lambda b, i: (0, 0)),
        ],
        out_shape=[
            jax.ShapeDtypeStruct((nb, N, O), jnp.float32),
            jax.ShapeDtypeStruct((1, O), jnp.float32),
            jax.ShapeDtypeStruct((1, O), jnp.float32),
        ],
    )(xt, fg, W)


def _norm_kernel(m_ref, hsum_ref, hsq_ref, g_ref, b_ref, o_ref, *, M, Dp):
    mean = hsum_ref[0] / M
    var = hsq_ref[0] / M - mean * mean
    h = ((m_ref[0] - mean[None, :]) / jnp.sqrt(var + EPS)[None, :]
         * g_ref[...][None, :] + b_ref[...][None, :])
    h = jnp.where(h >= 0, h, 0.2 * h)
    O = h.shape[-1]
    if Dp > O:
        h = jnp.pad(h, ((0, 0), (0, Dp - O)))
    o_ref[0] = h


def _bn_lrelu(m, hsum, hsq, g, b, Dp=None):
    O = m.shape[-1]
    Dp = Dp or O
    return pl.pallas_call(
        functools.partial(_norm_kernel, M=float(B * N * K), Dp=Dp),
        grid=(B,),
        in_specs=[
            pl.BlockSpec((1, N, O), lambda i: (i, 0, 0)),
            pl.BlockSpec((1, O), lambda i: (0, 0)),
            pl.BlockSpec((1, O), lambda i: (0, 0)),
            pl.BlockSpec((O,), lambda i: (0,)),
            pl.BlockSpec((O,), lambda i: (0,)),
        ],
        out_specs=pl.BlockSpec((1, N, Dp), lambda i: (i, 0, 0)),
        out_shape=jax.ShapeDtypeStruct((B, N, Dp), jnp.float32),
    )(m, hsum, hsq, g, b)


_NW = 32
_CH = 128


def _gather(xt, idx_flat):
    Dp = xt.shape[-1]
    nb = idx_flat.shape[0]
    rows = nb * N * K
    per = rows // _NW
    table = xt.reshape(nb * N, Dp)
    idx1 = idx_flat.reshape(rows)
    mesh = plsc.VectorSubcoreMesh(core_axis_name="c", subcore_axis_name="s")

    @functools.partial(
        pl.kernel, mesh=mesh,
        compiler_params=pltpu.CompilerParams(use_tc_tiling_on_sc=False),
        out_type=jax.ShapeDtypeStruct((rows, Dp), jnp.float32),
        scratch_types=[
            pltpu.VMEM((_CH,), jnp.int32),
            pltpu.VMEM((_CH, Dp), jnp.float32),
            pltpu.VMEM((_CH,), jnp.int32),
            pltpu.VMEM((_CH, Dp), jnp.float32),
            pltpu.SemaphoreType.DMA,
            pltpu.SemaphoreType.DMA,
        ],
    )
    def gk(table_hbm, idx_hbm, out_hbm, idx_a, buf_a, idx_b, buf_b, sem_a,
           sem_b):
        wid = lax.axis_index("s") * 2 + lax.axis_index("c")
        base = wid * per
        nch = per // (2 * _CH)

        def body(i, _):
            off = base + i * 2 * _CH
            pltpu.sync_copy(idx_hbm.at[pl.ds(off, _CH)], idx_a)
            cp_a = pltpu.async_copy(table_hbm.at[idx_a], buf_a, sem_a)
            pltpu.sync_copy(idx_hbm.at[pl.ds(off + _CH, _CH)], idx_b)
            cp_b = pltpu.async_copy(table_hbm.at[idx_b], buf_b, sem_b)
            cp_a.wait()
            pltpu.sync_copy(buf_a, out_hbm.at[pl.ds(off, _CH)])
            cp_b.wait()
            pltpu.sync_copy(buf_b, out_hbm.at[pl.ds(off + _CH, _CH)])
            return 0

        lax.fori_loop(0, nch, body, 0)

    return gk(table, idx1).reshape(nb, N, K, Dp)


def _final_kernel(x1_ref, x2_ref, x3_ref, x4_ref, w_ref,
                  hsum_ref, hsq_ref, hmax_ref):
    i = pl.program_id(0)
    w = w_ref[...]
    h = jax.lax.dot_general(
        x1_ref[0], w[:, 0:64], (((1,), (1,)), ((), ())),
        preferred_element_type=jnp.float32)
    h += jax.lax.dot_general(
        x2_ref[0], w[:, 64:128], (((1,), (1,)), ((), ())),
        preferred_element_type=jnp.float32)
    h += jax.lax.dot_general(
        x3_ref[0], w[:, 128:256], (((1,), (1,)), ((), ())),
        preferred_element_type=jnp.float32)
    h += jax.lax.dot_general(
        x4_ref[0], w[:, 256:512], (((1,), (1,)), ((), ())),
        preferred_element_type=jnp.float32)
    s = jnp.sum(h, axis=0)
    q = jnp.sum(h * h, axis=0)
    m = jnp.max(h, axis=0)

    @pl.when(i == 0)
    def _():
        hsum_ref[...] = jnp.zeros_like(hsum_ref)
        hsq_ref[...] = jnp.zeros_like(hsq_ref)

    hsum_ref[...] += s[None, :]
    hsq_ref[...] += q[None, :]
    hmax_ref[pl.ds(i, 1), :] = m[None, :]


def _final_norm_kernel(hsum_ref, hsq_ref, hmax_ref, g_ref, b_ref, o_ref):
    M = float(B * N)
    mean = hsum_ref[0] / M
    var = hsq_ref[0] / M - mean * mean
    h = ((hmax_ref[...] - mean[None, :]) / jnp.sqrt(var + EPS)[None, :]
         * g_ref[...][None, :] + b_ref[...][None, :])
    o_ref[...] = jnp.where(h >= 0, h, 0.2 * h)


def _final_stage(x1, x2, x3, x4, W5, g5, b5):
    O = W5.shape[0]
    hsum, hsq, hmax = pl.pallas_call(
        _final_kernel,
        grid=(B,),
        in_specs=[
            pl.BlockSpec((1, N, 64), lambda i: (i, 0, 0)),
            pl.BlockSpec((1, N, 64), lambda i: (i, 0, 0)),
            pl.BlockSpec((1, N, 128), lambda i: (i, 0, 0)),
            pl.BlockSpec((1, N, 256), lambda i: (i, 0, 0)),
            pl.BlockSpec((O, 512), lambda i: (0, 0)),
        ],
        out_specs=[
            pl.BlockSpec((1, O), lambda i: (0, 0)),
            pl.BlockSpec((1, O), lambda i: (0, 0)),
            pl.BlockSpec((B, O), lambda i: (0, 0)),
        ],
        out_shape=[
            jax.ShapeDtypeStruct((1, O), jnp.float32),
            jax.ShapeDtypeStruct((1, O), jnp.float32),
            jax.ShapeDtypeStruct((B, O), jnp.float32),
        ],
    )(x1, x2, x3, x4, W5)
    return pl.pallas_call(
        _final_norm_kernel,
        out_shape=jax.ShapeDtypeStruct((B, O), jnp.float32),
    )(hsum, hsq, hmax, g5, b5)


def _layer(prev, W, waves=2):
    nb = B // waves
    ms, hsums, hsqs, xts = [], [], [], []
    for w in range(waves):
        if prev[0] == "raw":
            xt_w = jax.lax.slice_in_dim(prev[1], w * nb, (w + 1) * nb, axis=0)
            idx = _knn_raw(prev[1], w * nb, nb)
        else:
            _, m_in, hs_in, hq_in, g_in, b_in = prev
            idx, xt_w = _knn_norm(m_in, hs_in, hq_in, g_in, b_in, w * nb, nb)
        fg = _gather(xt_w, idx)
        m, hsum, hsq = _edge_mm(xt_w, fg, W, 0, nb)
        ms.append(m)
        hsums.append(hsum)
        hsqs.append(hsq)
        xts.append(xt_w)
    m = jnp.concatenate(ms, axis=0)
    hsum = functools.reduce(jnp.add, hsums)
    hsq = functools.reduce(jnp.add, hsqs)
    xt_full = jnp.concatenate(xts, axis=0) if prev[0] == "norm" else None
    return m, hsum, hsq, xt_full


def kernel(x, W1, W2, W3, W4, W5, g1, b1, g2, b2, g3, b3, g4, b4, g5, b5):
    xt0 = jnp.pad(jnp.transpose(x, (0, 2, 1)), ((0, 0), (0, 0), (0, 13)))
    m1, s1, q1, _ = _layer(("raw", xt0), W1)
    m2, s2, q2, x1 = _layer(("norm", m1, s1, q1, g1, b1), W2)
    m3, s3, q3, x2 = _layer(("norm", m2, s2, q2, g2, b2), W3)
    m4, s4, q4, x3 = _layer(("norm", m3, s3, q3, g3, b3), W4)
    x4 = _bn_lrelu(m4, s4, q4, g4, b4)
    return _final_stage(x1, x2, x3, x4, W5, g5, b5)
